# Initial kernel scaffold; baseline (speedup 1.0000x reference)
import jax
import jax.numpy as jnp
from jax.experimental import pallas as pl


def kernel(tp_inputs, table, gamma, beta):
    raise NotImplementedError("write your pallas kernel here")


# speedup vs baseline: 1.0154x; 1.0154x over previous
import functools

import jax
import jax.numpy as jnp
from jax import lax
from jax.experimental import pallas as pl
from jax.experimental.pallas import tpu as pltpu
from jax.experimental.pallas import tpu_sc as plsc

_VOCAB = 100000
_D = 1024
_B = 4
_S = 2048
_N = _B * _S
_EPS = 1e-5

_NC = 2
_NS = 16
_NW = _NC * _NS
_PER_W = _N // _NW
_CHUNK = 64


def _sc_gather(ids, table):
    mesh = plsc.VectorSubcoreMesh(core_axis_name="c", subcore_axis_name="s")

    @functools.partial(
        pl.kernel,
        mesh=mesh,
        out_type=jax.ShapeDtypeStruct((_N, _D), jnp.float32),
        scratch_types=[
            pltpu.VMEM((_PER_W,), jnp.int32),
            pltpu.VMEM((_CHUNK, _D), jnp.float32),
            pltpu.SemaphoreType.DMA,
        ],
    )
    def gather_kernel(ids_hbm, table_hbm, out_hbm, idx_v, rows_v, sem):
        wid = lax.axis_index("s") * _NC + lax.axis_index("c")
        base = wid * _PER_W
        pltpu.sync_copy(ids_hbm.at[pl.ds(base, _PER_W)], idx_v)
        for c in range(_PER_W // _CHUNK):
            pltpu.async_copy(
                table_hbm.at[idx_v.at[pl.ds(c * _CHUNK, _CHUNK)]], rows_v, sem
            ).wait()
            pltpu.sync_copy(rows_v, out_hbm.at[pl.ds(base + c * _CHUNK, _CHUNK)])

    return gather_kernel(ids, table)


_LN_BLK = 256


def _ln_body(x_ref, g_ref, b_ref, o_ref):
    x = x_ref[...]
    mu = jnp.mean(x, axis=-1, keepdims=True)
    xc = x - mu
    var = jnp.mean(xc * xc, axis=-1, keepdims=True)
    o_ref[...] = xc * lax.rsqrt(var + _EPS) * g_ref[...] + b_ref[...]


def _tc_layernorm(x, gamma, beta):
    return pl.pallas_call(
        _ln_body,
        grid=(_N // _LN_BLK,),
        in_specs=[
            pl.BlockSpec((_LN_BLK, _D), lambda i: (i, 0)),
            pl.BlockSpec((1, _D), lambda i: (0, 0)),
            pl.BlockSpec((1, _D), lambda i: (0, 0)),
        ],
        out_specs=pl.BlockSpec((_LN_BLK, _D), lambda i: (i, 0)),
        out_shape=jax.ShapeDtypeStruct((_N, _D), jnp.float32),
    )(x, gamma.reshape(1, _D), beta.reshape(1, _D))


def kernel(tp_inputs, table, gamma, beta):
    ids = tp_inputs[..., 0].reshape(_N)
    mask = tp_inputs[..., 1].astype(jnp.float32)
    rows = _sc_gather(ids, table)
    hidden = _tc_layernorm(rows, gamma, beta)
    return hidden.reshape(_B, _S, _D), mask


# speedup vs baseline: 1.0234x; 1.0079x over previous
import functools

import jax
import jax.numpy as jnp
from jax import lax
from jax.experimental import pallas as pl
from jax.experimental.pallas import tpu as pltpu
from jax.experimental.pallas import tpu_sc as plsc

_VOCAB = 100000
_D = 1024
_B = 4
_S = 2048
_N = _B * _S
_EPS = 1e-5

_NC = 2
_NS = 16
_NW = _NC * _NS
_PER_W = _N // _NW
_CHUNK = 32
_NCHUNK = _PER_W // _CHUNK


def _sc_gather(ids, table):
    mesh = plsc.VectorSubcoreMesh(core_axis_name="c", subcore_axis_name="s")

    @functools.partial(
        pl.kernel,
        mesh=mesh,
        out_type=jax.ShapeDtypeStruct((_N, _D), jnp.float32),
        scratch_types=[
            pltpu.VMEM((_PER_W,), jnp.int32),
            pltpu.VMEM((_CHUNK, _D), jnp.float32),
            pltpu.VMEM((_CHUNK, _D), jnp.float32),
            pltpu.SemaphoreType.DMA,
            pltpu.SemaphoreType.DMA,
            pltpu.SemaphoreType.DMA,
            pltpu.SemaphoreType.DMA,
        ],
    )
    def gather_kernel(ids_hbm, table_hbm, out_hbm, idx_v, r0, r1, g0, g1, w0, w1):
        wid = lax.axis_index("s") * _NC + lax.axis_index("c")
        base = wid * _PER_W
        pltpu.sync_copy(ids_hbm.at[pl.ds(base, _PER_W)], idx_v)
        bufs, gsems, wsems = (r0, r1), (g0, g1), (w0, w1)

        def start_gather(c):
            b = c % 2
            return pltpu.async_copy(
                table_hbm.at[idx_v.at[pl.ds(c * _CHUNK, _CHUNK)]], bufs[b], gsems[b]
            )

        gcopy = [None] * _NCHUNK
        wcopy = [None] * _NCHUNK
        gcopy[0] = start_gather(0)
        for c in range(_NCHUNK):
            b = c % 2
            if c + 1 < _NCHUNK:
                if c >= 1:
                    wcopy[c - 1].wait()
                gcopy[c + 1] = start_gather(c + 1)
            gcopy[c].wait()
            wcopy[c] = pltpu.async_copy(
                bufs[b], out_hbm.at[pl.ds(base + c * _CHUNK, _CHUNK)], wsems[b]
            )
        wcopy[_NCHUNK - 2].wait()
        wcopy[_NCHUNK - 1].wait()

    return gather_kernel(ids, table)


_LN_BLK = 256


def _ln_body(x_ref, g_ref, b_ref, o_ref):
    x = x_ref[...]
    mu = jnp.mean(x, axis=-1, keepdims=True)
    xc = x - mu
    var = jnp.mean(xc * xc, axis=-1, keepdims=True)
    o_ref[...] = xc * lax.rsqrt(var + _EPS) * g_ref[...] + b_ref[...]


def _tc_layernorm(x, gamma, beta):
    return pl.pallas_call(
        _ln_body,
        grid=(_N // _LN_BLK,),
        in_specs=[
            pl.BlockSpec((_LN_BLK, _D), lambda i: (i, 0)),
            pl.BlockSpec((1, _D), lambda i: (0, 0)),
            pl.BlockSpec((1, _D), lambda i: (0, 0)),
        ],
        out_specs=pl.BlockSpec((_LN_BLK, _D), lambda i: (i, 0)),
        out_shape=jax.ShapeDtypeStruct((_N, _D), jnp.float32),
    )(x, gamma.reshape(1, _D), beta.reshape(1, _D))


def kernel(tp_inputs, table, gamma, beta):
    ids = tp_inputs[..., 0].reshape(_N)
    mask = tp_inputs[..., 1].astype(jnp.float32)
    rows = _sc_gather(ids, table)
    hidden = _tc_layernorm(rows, gamma, beta)
    return hidden.reshape(_B, _S, _D), mask


# speedup vs baseline: 1.1644x; 1.1377x over previous
import functools

import jax
import jax.numpy as jnp
from jax import lax
from jax.experimental import pallas as pl
from jax.experimental.pallas import tpu as pltpu
from jax.experimental.pallas import tpu_sc as plsc

_VOCAB = 100000
_D = 1024
_B = 4
_S = 2048
_N = _B * _S
_EPS = 1e-5

_NC = 2
_NS = 16
_NW = _NC * _NS
_PER_W = _N // _NW
_CHUNK = 32
_NCHUNK = _PER_W // _CHUNK


def _sc_gather(ids, table):
    mesh = plsc.VectorSubcoreMesh(core_axis_name="c", subcore_axis_name="s")

    @functools.partial(
        pl.kernel,
        mesh=mesh,
        out_type=jax.ShapeDtypeStruct((_N, _D), jnp.float32),
        scratch_types=[
            pltpu.VMEM((_PER_W,), jnp.int32),
            pltpu.VMEM((_CHUNK, _D), jnp.float32),
            pltpu.VMEM((_CHUNK, _D), jnp.float32),
            pltpu.SemaphoreType.DMA,
            pltpu.SemaphoreType.DMA,
            pltpu.SemaphoreType.DMA,
            pltpu.SemaphoreType.DMA,
        ],
    )
    def gather_kernel(ids_hbm, table_hbm, out_hbm, idx_v, r0, r1, g0, g1, w0, w1):
        wid = lax.axis_index("s") * _NC + lax.axis_index("c")
        base = wid * _PER_W
        pltpu.sync_copy(ids_hbm.at[pl.ds(base, _PER_W)], idx_v)
        bufs, gsems, wsems = (r0, r1), (g0, g1), (w0, w1)

        def start_gather(c):
            b = c % 2
            return pltpu.async_copy(
                table_hbm.at[idx_v.at[pl.ds(c * _CHUNK, _CHUNK)]], bufs[b], gsems[b]
            )

        gcopy = [None] * _NCHUNK
        wcopy = [None] * _NCHUNK
        gcopy[0] = start_gather(0)
        for c in range(_NCHUNK):
            b = c % 2
            if c + 1 < _NCHUNK:
                if c >= 1:
                    wcopy[c - 1].wait()
                gcopy[c + 1] = start_gather(c + 1)
            gcopy[c].wait()
            wcopy[c] = pltpu.async_copy(
                bufs[b], out_hbm.at[pl.ds(base + c * _CHUNK, _CHUNK)], wsems[b]
            )
        wcopy[_NCHUNK - 2].wait()
        wcopy[_NCHUNK - 1].wait()

    return gather_kernel(ids, table)


_LN_BLK = 512


def _ln_body(x_ref, g_ref, b_ref, o_ref):
    x = x_ref[...]
    mu = jnp.mean(x, axis=-1, keepdims=True)
    xc = x - mu
    var = jnp.mean(xc * xc, axis=-1, keepdims=True)
    o_ref[...] = xc * lax.rsqrt(var + _EPS) * g_ref[...] + b_ref[...]


def _tc_layernorm(x, gamma, beta):
    return pl.pallas_call(
        _ln_body,
        grid=(_N // _LN_BLK,),
        in_specs=[
            pl.BlockSpec((_LN_BLK, _D), lambda i: (i, 0)),
            pl.BlockSpec((1, _D), lambda i: (0, 0)),
            pl.BlockSpec((1, _D), lambda i: (0, 0)),
        ],
        out_specs=pl.BlockSpec((_LN_BLK, _D), lambda i: (i, 0)),
        out_shape=jax.ShapeDtypeStruct((_N, _D), jnp.float32),
    )(x, gamma.reshape(1, _D), beta.reshape(1, _D))


def kernel(tp_inputs, table, gamma, beta):
    ids = tp_inputs[..., 0].reshape(_N)
    mask = tp_inputs[..., 1].astype(jnp.float32)
    rows = _sc_gather(ids, table)
    hidden = _tc_layernorm(rows, gamma, beta)
    return hidden.reshape(_B, _S, _D), mask


# speedup vs baseline: 1.2430x; 1.0675x over previous
import functools

import jax
import jax.numpy as jnp
from jax import lax
from jax.experimental import pallas as pl
from jax.experimental.pallas import tpu as pltpu
from jax.experimental.pallas import tpu_sc as plsc

_VOCAB = 100000
_D = 1024
_B = 4
_S = 2048
_N = _B * _S
_EPS = 1e-5

_NC = 2
_NS = 16
_NW = _NC * _NS
_PER_W = _N // _NW
_CHUNK = 32
_NCHUNK = _PER_W // _CHUNK


def _sc_gather(ids, table):
    mesh = plsc.VectorSubcoreMesh(core_axis_name="c", subcore_axis_name="s")

    @functools.partial(
        pl.kernel,
        mesh=mesh,
        out_type=jax.ShapeDtypeStruct((_N, _D), jnp.float32),
        scratch_types=[
            pltpu.VMEM((_PER_W,), jnp.int32),
            pltpu.VMEM((_CHUNK, _D), jnp.float32),
            pltpu.VMEM((_CHUNK, _D), jnp.float32),
            pltpu.SemaphoreType.DMA,
            pltpu.SemaphoreType.DMA,
            pltpu.SemaphoreType.DMA,
            pltpu.SemaphoreType.DMA,
        ],
    )
    def gather_kernel(ids_hbm, table_hbm, out_hbm, idx_v, r0, r1, g0, g1, w0, w1):
        wid = lax.axis_index("s") * _NC + lax.axis_index("c")
        base = wid * _PER_W
        pltpu.sync_copy(ids_hbm.at[pl.ds(base, _PER_W)], idx_v)
        bufs, gsems, wsems = (r0, r1), (g0, g1), (w0, w1)

        def start_gather(c):
            b = c % 2
            return pltpu.async_copy(
                table_hbm.at[idx_v.at[pl.ds(c * _CHUNK, _CHUNK)]], bufs[b], gsems[b]
            )

        gcopy = [None] * _NCHUNK
        wcopy = [None] * _NCHUNK
        gcopy[0] = start_gather(0)
        for c in range(_NCHUNK):
            b = c % 2
            if c + 1 < _NCHUNK:
                if c >= 1:
                    wcopy[c - 1].wait()
                gcopy[c + 1] = start_gather(c + 1)
            gcopy[c].wait()
            wcopy[c] = pltpu.async_copy(
                bufs[b], out_hbm.at[pl.ds(base + c * _CHUNK, _CHUNK)], wsems[b]
            )
        wcopy[_NCHUNK - 2].wait()
        wcopy[_NCHUNK - 1].wait()

    return gather_kernel(ids, table)


_LN_BLK = 1024


def _ln_body(x_ref, g_ref, b_ref, o_ref):
    x = x_ref[...]
    mu = jnp.mean(x, axis=-1, keepdims=True)
    xc = x - mu
    var = jnp.mean(xc * xc, axis=-1, keepdims=True)
    o_ref[...] = xc * lax.rsqrt(var + _EPS) * g_ref[...] + b_ref[...]


def _tc_layernorm(x, gamma, beta):
    return pl.pallas_call(
        _ln_body,
        grid=(_N // _LN_BLK,),
        in_specs=[
            pl.BlockSpec((_LN_BLK, _D), lambda i: (i, 0)),
            pl.BlockSpec((1, _D), lambda i: (0, 0)),
            pl.BlockSpec((1, _D), lambda i: (0, 0)),
        ],
        out_specs=pl.BlockSpec((_LN_BLK, _D), lambda i: (i, 0)),
        out_shape=jax.ShapeDtypeStruct((_N, _D), jnp.float32),
    )(x, gamma.reshape(1, _D), beta.reshape(1, _D))


def kernel(tp_inputs, table, gamma, beta):
    ids = tp_inputs[..., 0].reshape(_N)
    mask = tp_inputs[..., 1].astype(jnp.float32)
    rows = _sc_gather(ids, table)
    hidden = _tc_layernorm(rows, gamma, beta)
    return hidden.reshape(_B, _S, _D), mask
